# Initial kernel scaffold; baseline (speedup 1.0000x reference)
import jax
import jax.numpy as jnp
from jax.experimental import pallas as pl


def kernel(features, edge_index, edge_weight, W1, W2):
    raise NotImplementedError("write your pallas kernel here")


# speedup vs baseline: 3.9130x; 3.9130x over previous
import functools

import jax
import jax.numpy as jnp
from jax import lax
from jax.experimental import pallas as pl
from jax.experimental.pallas import tpu as pltpu
from jax.experimental.pallas import tpu_sc as plsc

_N = 10000
_E = 320000
_NC = 2
_NS = 16
_NW = _NC * _NS
_B = 80
_EPT = _E // _NW
_ROWS_PT = 624
_TAIL_OFF = _ROWS_PT * _NS
_TAIL = _N - _TAIL_OFF


def _make_spmm(D):
  mesh = plsc.VectorSubcoreMesh(core_axis_name="c", subcore_axis_name="s")
  n_chunks = _EPT // _B

  @functools.partial(
      pl.kernel,
      out_type=jax.ShapeDtypeStruct((_NC, _N, D), jnp.float32),
      mesh=mesh,
      compiler_params=pltpu.CompilerParams(use_tc_tiling_on_sc=False),
      scratch_types=[
          pltpu.VMEM((_B,), jnp.int32),
          pltpu.VMEM((_B,), jnp.int32),
          pltpu.VMEM((_B,), jnp.float32),
          pltpu.VMEM((_B, D), jnp.float32),
          pltpu.VMEM_SHARED((_N, D), jnp.float32),
          pltpu.SemaphoreType.DMA,
      ],
  )
  def spmm(x_hbm, src_hbm, dst_hbm, w_hbm, zeros_hbm, out_hbm,
           src_v, dst_v, w_v, rows_v, acc, sem):
    c = lax.axis_index("c")
    s = lax.axis_index("s")
    pltpu.sync_copy(zeros_hbm.at[pl.ds(s * _ROWS_PT, _ROWS_PT)],
                    acc.at[pl.ds(s * _ROWS_PT, _ROWS_PT)])

    @pl.when(s == 0)
    def _zero_tail():
      pltpu.sync_copy(zeros_hbm.at[pl.ds(_TAIL_OFF, _TAIL)],
                      acc.at[pl.ds(_TAIL_OFF, _TAIL)])

    plsc.subcore_barrier()

    base = (c * _NS + s) * _EPT

    def chunk(i, carry):
      off = base + i * _B
      pltpu.sync_copy(src_hbm.at[pl.ds(off, _B)], src_v)
      pltpu.sync_copy(dst_hbm.at[pl.ds(off, _B)], dst_v)
      pltpu.sync_copy(w_hbm.at[pl.ds(off, _B)], w_v)
      pltpu.async_copy(x_hbm.at[src_v], rows_v, sem).wait()

      def scale(g, carry2):
        wvec = w_v[pl.ds(g * 16, 16)]
        for j in range(16):
          wj = wvec[j]
          k = g * 16 + j
          for d in range(D // 16):
            sl = pl.ds(d * 16, 16)
            rows_v[k, sl] = rows_v[k, sl] * wj
        return carry2

      lax.fori_loop(0, _B // 16, scale, 0)
      pltpu.sync_copy(rows_v, acc.at[dst_v], add=True)
      return carry

    lax.fori_loop(0, n_chunks, chunk, 0)
    plsc.subcore_barrier()
    pltpu.sync_copy(acc.at[pl.ds(s * _ROWS_PT, _ROWS_PT)],
                    out_hbm.at[c, pl.ds(s * _ROWS_PT, _ROWS_PT)])

    @pl.when(s == 0)
    def _write_tail():
      pltpu.sync_copy(acc.at[pl.ds(_TAIL_OFF, _TAIL)],
                      out_hbm.at[c, pl.ds(_TAIL_OFF, _TAIL)])

  return spmm


_spmm_128 = _make_spmm(128)
_spmm_64 = _make_spmm(64)


def _tc_matmul(x, w):
  def body(x_ref, w_ref, o_ref):
    o_ref[...] = jnp.dot(x_ref[...], w_ref[...],
                         preferred_element_type=jnp.float32)

  return pl.pallas_call(
      body,
      out_shape=jax.ShapeDtypeStruct((x.shape[0], w.shape[1]), jnp.float32),
  )(x, w)


def _tc_relu_add_matmul(p, w):
  def body(p_ref, w_ref, o_ref):
    h = jnp.maximum(p_ref[0] + p_ref[1], 0.0)
    o_ref[...] = jnp.dot(h, w_ref[...], preferred_element_type=jnp.float32)

  return pl.pallas_call(
      body,
      out_shape=jax.ShapeDtypeStruct((p.shape[1], w.shape[1]), jnp.float32),
  )(p, w)


def _tc_add(p):
  def body(p_ref, o_ref):
    o_ref[...] = p_ref[0] + p_ref[1]

  return pl.pallas_call(
      body,
      out_shape=jax.ShapeDtypeStruct(p.shape[1:], jnp.float32),
  )(p)


def kernel(features, edge_index, edge_weight, W1, W2):
  src = edge_index[0].astype(jnp.int32)
  dst = edge_index[1].astype(jnp.int32)
  ew = edge_weight.astype(jnp.float32)
  z128 = jnp.zeros((_N, 128), jnp.float32)
  z64 = jnp.zeros((_N, 64), jnp.float32)

  xw = _tc_matmul(features, W1)
  p1 = _spmm_128(xw, src, dst, ew, z128)
  hw = _tc_relu_add_matmul(p1, W2)
  p2 = _spmm_64(hw, src, dst, ew, z64)
  return _tc_add(p2)


# speedup vs baseline: 7.8381x; 2.0031x over previous
import functools

import jax
import jax.numpy as jnp
from jax import lax
from jax.experimental import pallas as pl
from jax.experimental.pallas import tpu as pltpu
from jax.experimental.pallas import tpu_sc as plsc

_N = 10000
_E = 320000
_NC = 2
_NS = 16
_NW = _NC * _NS
_B = 80
_EPT = _E // _NW
_ROWS_PT = 624
_TAIL_OFF = _ROWS_PT * _NS
_TAIL = _N - _TAIL_OFF


def _make_spmm(D):
  mesh = plsc.VectorSubcoreMesh(core_axis_name="c", subcore_axis_name="s")
  n_chunks = _EPT // _B
  n_pairs = (n_chunks - 1) // 2

  @functools.partial(
      pl.kernel,
      out_type=jax.ShapeDtypeStruct((_NC, _N, D), jnp.float32),
      mesh=mesh,
      compiler_params=pltpu.CompilerParams(use_tc_tiling_on_sc=False),
      scratch_types=[
          pltpu.VMEM((n_chunks, _B), jnp.int32),
          pltpu.VMEM((n_chunks, _B), jnp.int32),
          pltpu.VMEM((n_chunks, _B), jnp.float32),
          pltpu.VMEM((_B, D), jnp.float32),
          pltpu.VMEM((_B, D), jnp.float32),
          pltpu.VMEM_SHARED((_N, D), jnp.float32),
          pltpu.SemaphoreType.DMA,
          pltpu.SemaphoreType.DMA,
          pltpu.SemaphoreType.DMA,
      ],
  )
  def spmm(x_hbm, src_hbm, dst_hbm, w_hbm, zeros_hbm, out_hbm,
           src_v, dst_v, w_v, rows0, rows1, acc, g0, g1, gi):
    c = lax.axis_index("c")
    s = lax.axis_index("s")
    wid = c * _NS + s
    ci0 = pltpu.async_copy(src_hbm.at[wid], src_v, gi)
    ci1 = pltpu.async_copy(dst_hbm.at[wid], dst_v, gi)
    ci2 = pltpu.async_copy(w_hbm.at[wid], w_v, gi)
    pltpu.sync_copy(zeros_hbm.at[pl.ds(s * _ROWS_PT, _ROWS_PT)],
                    acc.at[pl.ds(s * _ROWS_PT, _ROWS_PT)])

    @pl.when(s == 0)
    def _zero_tail():
      pltpu.sync_copy(zeros_hbm.at[pl.ds(_TAIL_OFF, _TAIL)],
                      acc.at[pl.ds(_TAIL_OFF, _TAIL)])

    ci0.wait()
    ci1.wait()
    ci2.wait()
    plsc.subcore_barrier()

    def gather_start(i, buf, sem):
      pltpu.async_copy(x_hbm.at[src_v.at[i]], buf, sem)

    def gather_wait(i, buf, sem):
      pltpu.make_async_copy(x_hbm.at[src_v.at[i]], buf, sem).wait()

    def scale(buf, i):
      def grp(g, carry):
        wvec = w_v[i, pl.ds(g * 16, 16)]
        for j in range(16):
          wj = wvec[j]
          k = g * 16 + j
          for d in range(D // 16):
            sl = pl.ds(d * 16, 16)
            buf[k, sl] = buf[k, sl] * wj
        return carry

      lax.fori_loop(0, _B // 16, grp, 0)

    def scatter(i, buf):
      pltpu.sync_copy(buf, acc.at[dst_v.at[i]], add=True)

    gather_start(0, rows0, g0)
    gather_start(1, rows1, g1)

    def pair(p, carry):
      i0 = 2 * p
      gather_wait(i0, rows0, g0)
      scale(rows0, i0)
      scatter(i0, rows0)
      gather_start(i0 + 2, rows0, g0)
      gather_wait(i0 + 1, rows1, g1)
      scale(rows1, i0 + 1)
      scatter(i0 + 1, rows1)

      @pl.when(p < n_pairs - 1)
      def _next():
        gather_start(i0 + 3, rows1, g1)

      return carry

    lax.fori_loop(0, n_pairs, pair, 0)
    last = n_chunks - 1
    gather_wait(last, rows0, g0)
    scale(rows0, last)
    scatter(last, rows0)

    plsc.subcore_barrier()
    pltpu.sync_copy(acc.at[pl.ds(s * _ROWS_PT, _ROWS_PT)],
                    out_hbm.at[c, pl.ds(s * _ROWS_PT, _ROWS_PT)])

    @pl.when(s == 0)
    def _write_tail():
      pltpu.sync_copy(acc.at[pl.ds(_TAIL_OFF, _TAIL)],
                      out_hbm.at[c, pl.ds(_TAIL_OFF, _TAIL)])

  return spmm


_spmm_128 = _make_spmm(128)
_spmm_64 = _make_spmm(64)


def _tc_matmul(x, w):
  def body(x_ref, w_ref, o_ref):
    o_ref[...] = jnp.dot(x_ref[...], w_ref[...],
                         preferred_element_type=jnp.float32)

  return pl.pallas_call(
      body,
      out_shape=jax.ShapeDtypeStruct((x.shape[0], w.shape[1]), jnp.float32),
  )(x, w)


def _tc_relu_add_matmul(p, w):
  def body(p_ref, w_ref, o_ref):
    h = jnp.maximum(p_ref[0] + p_ref[1], 0.0)
    o_ref[...] = jnp.dot(h, w_ref[...], preferred_element_type=jnp.float32)

  return pl.pallas_call(
      body,
      out_shape=jax.ShapeDtypeStruct((p.shape[1], w.shape[1]), jnp.float32),
  )(p, w)


def _tc_add(p):
  def body(p_ref, o_ref):
    o_ref[...] = p_ref[0] + p_ref[1]

  return pl.pallas_call(
      body,
      out_shape=jax.ShapeDtypeStruct(p.shape[1:], jnp.float32),
  )(p)


def kernel(features, edge_index, edge_weight, W1, W2):
  n_chunks = _EPT // _B
  src = edge_index[0].astype(jnp.int32).reshape(_NW, n_chunks, _B)
  dst = edge_index[1].astype(jnp.int32).reshape(_NW, n_chunks, _B)
  ew = edge_weight.astype(jnp.float32).reshape(_NW, n_chunks, _B)
  z128 = jnp.zeros((_N, 128), jnp.float32)
  z64 = jnp.zeros((_N, 64), jnp.float32)

  xw = _tc_matmul(features, W1)
  p1 = _spmm_128(xw, src, dst, ew, z128)
  hw = _tc_relu_add_matmul(p1, W2)
  p2 = _spmm_64(hw, src, dst, ew, z64)
  return _tc_add(p2)
